# Initial kernel scaffold; baseline (speedup 1.0000x reference)
import jax
import jax.numpy as jnp
from jax.experimental import pallas as pl


def kernel(fixed_values, refinable_params, refinable_idx):
    raise NotImplementedError("write your pallas kernel here")


# speedup vs baseline: 1.4655x; 1.4655x over previous
import jax
import jax.numpy as jnp
from jax import lax
from jax.experimental import pallas as pl
from jax.experimental.pallas import tpu as pltpu
from jax.experimental.pallas import tpu_sc as plsc

_NC = 2
_NS = 16
_NW = _NC * _NS

_IDX_BATCH = 128


def _tc_copy(x2d, block_rows):
    rows, cols = x2d.shape

    def body(src_ref, dst_ref):
        dst_ref[...] = src_ref[...]

    return pl.pallas_call(
        body,
        grid=(rows // block_rows,),
        in_specs=[pl.BlockSpec((block_rows, cols), lambda i: (i, 0))],
        out_specs=pl.BlockSpec((block_rows, cols), lambda i: (i, 0)),
        out_shape=jax.ShapeDtypeStruct((rows, cols), x2d.dtype),
    )(x2d)


def kernel(fixed_values, refinable_params, refinable_idx):
    n = fixed_values.shape[0]
    r = refinable_params.shape[0]

    cols = 1024
    rows = n // cols
    copied = _tc_copy(fixed_values.reshape(rows, cols), block_rows=512)
    copied = copied.reshape(n)

    per_w = r // _NW
    n_batches = per_w // _IDX_BATCH

    idx3 = refinable_idx.reshape(_NW, n_batches, _IDX_BATCH)
    prm3 = refinable_params.reshape(_NW, n_batches, _IDX_BATCH)

    mesh = plsc.VectorSubcoreMesh(
        core_axis_name="c", subcore_axis_name="s",
        num_cores=_NC, num_subcores=_NS,
    )

    @pl.kernel(
        mesh=mesh,
        scratch_types=[
            pltpu.VMEM((n_batches, _IDX_BATCH), jnp.int32),
            pltpu.VMEM((n_batches, _IDX_BATCH), jnp.float32),
            pltpu.SemaphoreType.DMA,
        ],
    )
    def sc_scatter(out_hbm, idx_hbm, prm_hbm, idx_v, prm_v, sem):
        wid = lax.axis_index("s") * _NC + lax.axis_index("c")
        pltpu.sync_copy(idx_hbm.at[wid], idx_v)
        pltpu.sync_copy(prm_hbm.at[wid], prm_v)

        def fire(j, carry):
            pltpu.async_copy(prm_v.at[j], out_hbm.at[idx_v.at[j]], sem)
            return carry

        lax.fori_loop(0, n_batches, fire, 0)

        def drain(j, carry):
            pltpu.make_async_copy(
                prm_v.at[0], out_hbm.at[idx_v.at[0]], sem).wait()
            return carry

        lax.fori_loop(0, n_batches, drain, 0)

    out_ref = jax.new_ref(copied)
    sc_scatter(out_ref, idx3, prm3)
    return jax.freeze(out_ref)


# speedup vs baseline: 5.0205x; 3.4257x over previous
import jax
import jax.numpy as jnp
from jax import lax
from jax.experimental import pallas as pl
from jax.experimental.pallas import tpu as pltpu
from jax.experimental.pallas import tpu_sc as plsc

_NC = 2
_NS = 16
_NW = _NC * _NS

_PIECE = 32768
_CHUNK = 2048
_LANES = 16


def kernel(fixed_values, refinable_params, refinable_idx):
    n = fixed_values.shape[0]
    r = refinable_params.shape[0]
    n_pieces = n // _PIECE
    pieces_per_w = n_pieces // _NW

    boundaries = jnp.arange(n_pieces + 1, dtype=jnp.int32) * _PIECE
    bounds = jnp.searchsorted(refinable_idx, boundaries).astype(jnp.int32)
    n_bpad = ((n_pieces + 1 + _LANES + 7) // 8) * 8
    bounds = jnp.pad(bounds, (0, n_bpad - (n_pieces + 1)))
    idx_pad = jnp.pad(refinable_idx, (0, _CHUNK + 8),
                      constant_values=jnp.int32(2**31 - 1))
    prm_pad = jnp.pad(refinable_params, (0, _CHUNK + 8))

    mesh = plsc.VectorSubcoreMesh(
        core_axis_name="c", subcore_axis_name="s",
        num_cores=_NC, num_subcores=_NS,
    )

    @pl.kernel(
        mesh=mesh,
        out_type=jax.ShapeDtypeStruct((n,), jnp.float32),
        compiler_params=pltpu.CompilerParams(needs_layout_passes=False),
        scratch_types=[
            pltpu.VMEM((_PIECE,), jnp.float32),
            pltpu.VMEM((_CHUNK,), jnp.int32),
            pltpu.VMEM((_CHUNK,), jnp.float32),
            pltpu.VMEM((n_bpad,), jnp.int32),
        ],
    )
    def sc_merge(fixed_hbm, idx_hbm, prm_hbm, bounds_hbm, out_hbm,
                 buf, idx_v, prm_v, bounds_v):
        wid = lax.axis_index("s") * _NC + lax.axis_index("c")
        pltpu.sync_copy(bounds_hbm, bounds_v)

        def do_piece(k, carry):
            p = wid + k * _NW
            plo = p * _PIECE
            phi = plo + _PIECE
            pltpu.sync_copy(fixed_hbm.at[pl.ds(plo, _PIECE)], buf)

            bv = bounds_v[pl.ds(p, _LANES)]
            a = bv[0]
            b = bv[1]
            a_r = a & ~7
            n_chunks = (b - a_r + _CHUNK - 1) // _CHUNK

            def do_chunk(c, carry2):
                base = pl.multiple_of(a_r + c * _CHUNK, 8)
                pltpu.sync_copy(idx_hbm.at[pl.ds(base, _CHUNK)], idx_v)
                pltpu.sync_copy(prm_hbm.at[pl.ds(base, _CHUNK)], prm_v)
                rem = b - base
                n_vec = lax.min((rem + _LANES - 1) // _LANES,
                                _CHUNK // _LANES)

                def do_vec(v, carry3):
                    iv = idx_v[pl.ds(v * _LANES, _LANES)]
                    pv = prm_v[pl.ds(v * _LANES, _LANES)]
                    mask = (iv >= plo) & (iv < phi)
                    plsc.store_scatter(buf, [iv - plo], pv, mask=mask)
                    return carry3

                lax.fori_loop(0, n_vec, do_vec, 0)
                return carry2

            lax.fori_loop(0, n_chunks, do_chunk, 0)
            pltpu.sync_copy(buf, out_hbm.at[pl.ds(plo, _PIECE)])
            return carry

        lax.fori_loop(0, pieces_per_w, do_piece, 0)

    return sc_merge(fixed_values, idx_pad, prm_pad, bounds)


# speedup vs baseline: 9.7843x; 1.9489x over previous
import jax
import jax.numpy as jnp
from jax import lax
from jax.experimental import pallas as pl
from jax.experimental.pallas import tpu as pltpu
from jax.experimental.pallas import tpu_sc as plsc

_NC = 2
_NS = 16
_NW = _NC * _NS

_PIECE = 32768
_CHUNK = 2048
_LANES = 16


def kernel(fixed_values, refinable_params, refinable_idx):
    n = fixed_values.shape[0]
    r = refinable_params.shape[0]
    n_pieces = n // _PIECE
    pieces_per_w = n_pieces // _NW

    boundaries = jnp.arange(n_pieces + 1, dtype=jnp.int32) * _PIECE
    bounds = jnp.searchsorted(refinable_idx, boundaries,
                              method="compare_all").astype(jnp.int32)
    n_bpad = ((n_pieces + 1 + _LANES + 7) // 8) * 8
    bounds = jnp.pad(bounds, (0, n_bpad - (n_pieces + 1)))
    idx_pad = jnp.pad(refinable_idx, (0, _CHUNK + 8),
                      constant_values=jnp.int32(2**31 - 1))
    prm_pad = jnp.pad(refinable_params, (0, _CHUNK + 8))

    mesh = plsc.VectorSubcoreMesh(
        core_axis_name="c", subcore_axis_name="s",
        num_cores=_NC, num_subcores=_NS,
    )

    @pl.kernel(
        mesh=mesh,
        out_type=jax.ShapeDtypeStruct((n,), jnp.float32),
        compiler_params=pltpu.CompilerParams(needs_layout_passes=False),
        scratch_types=[
            pltpu.VMEM((_PIECE,), jnp.float32),
            pltpu.VMEM((_CHUNK,), jnp.int32),
            pltpu.VMEM((_CHUNK,), jnp.float32),
            pltpu.VMEM((n_bpad,), jnp.int32),
        ],
    )
    def sc_merge(fixed_hbm, idx_hbm, prm_hbm, bounds_hbm, out_hbm,
                 buf, idx_v, prm_v, bounds_v):
        wid = lax.axis_index("s") * _NC + lax.axis_index("c")
        pltpu.sync_copy(bounds_hbm, bounds_v)

        def do_piece(k, carry):
            p = wid + k * _NW
            plo = p * _PIECE
            phi = plo + _PIECE
            pltpu.sync_copy(fixed_hbm.at[pl.ds(plo, _PIECE)], buf)

            bv = bounds_v[pl.ds(p, _LANES)]
            a = bv[0]
            b = bv[1]
            a_r = a & ~7
            n_chunks = (b - a_r + _CHUNK - 1) // _CHUNK

            def do_chunk(c, carry2):
                base = pl.multiple_of(a_r + c * _CHUNK, 8)
                pltpu.sync_copy(idx_hbm.at[pl.ds(base, _CHUNK)], idx_v)
                pltpu.sync_copy(prm_hbm.at[pl.ds(base, _CHUNK)], prm_v)
                rem = b - base
                n_vec = lax.min((rem + _LANES - 1) // _LANES,
                                _CHUNK // _LANES)

                def do_vec(v, carry3):
                    iv = idx_v[pl.ds(v * _LANES, _LANES)]
                    pv = prm_v[pl.ds(v * _LANES, _LANES)]
                    mask = (iv >= plo) & (iv < phi)
                    plsc.store_scatter(buf, [iv - plo], pv, mask=mask)
                    return carry3

                lax.fori_loop(0, n_vec, do_vec, 0)
                return carry2

            lax.fori_loop(0, n_chunks, do_chunk, 0)
            pltpu.sync_copy(buf, out_hbm.at[pl.ds(plo, _PIECE)])
            return carry

        lax.fori_loop(0, pieces_per_w, do_piece, 0)

    return sc_merge(fixed_values, idx_pad, prm_pad, bounds)
